# Initial kernel scaffold; baseline (speedup 1.0000x reference)
import jax
import jax.numpy as jnp
from jax.experimental import pallas as pl


def kernel(x, edge_index, W1_root, W1_rel, b1, W2_root, W2_rel, b2, att_W):
    raise NotImplementedError("write your pallas kernel here")


# speedup vs baseline: 4.6757x; 4.6757x over previous
import functools

import jax
import jax.numpy as jnp
from jax import lax
from jax.experimental import pallas as pl
from jax.experimental.pallas import tpu as pltpu
from jax.experimental.pallas import tpu_sc as plsc

_NC = 2
_NS = 16
_NW = _NC * _NS
_L = 128


@functools.lru_cache(maxsize=None)
def _make_edge_agg(n_pad, kb, d):
    rpt = n_pad // _NS
    mesh = plsc.VectorSubcoreMesh(core_axis_name="c", subcore_axis_name="s",
                                  num_cores=_NC, num_subcores=_NS)

    @functools.partial(
        pl.kernel,
        out_type=(jax.ShapeDtypeStruct((_NC, n_pad, d), jnp.float32),
                  jax.ShapeDtypeStruct((_NW * n_pad,), jnp.float32)),
        mesh=mesh,
        compiler_params=pltpu.CompilerParams(needs_layout_passes=False),
        scratch_types=(
            pltpu.VMEM((kb, _L), jnp.int32),
            pltpu.VMEM((kb, _L), jnp.int32),
            pltpu.VMEM((_L, d), jnp.float32),
            pltpu.VMEM((n_pad,), jnp.float32),
            pltpu.VMEM_SHARED((n_pad, d), jnp.float32),
            pltpu.SemaphoreType.DMA,
        ),
    )
    def edge_agg(h_hbm, src_hbm, dst_hbm, zrow_hbm, zdeg_hbm,
                 acc_out, deg_out, src_v, dst_v, rows_v, deg_v, acc_sh, sem):
        c = lax.axis_index("c")
        s = lax.axis_index("s")
        wid = s * _NC + c
        pltpu.sync_copy(zrow_hbm.at[pl.ds(s * rpt, rpt)],
                        acc_sh.at[pl.ds(s * rpt, rpt)])
        pltpu.sync_copy(zdeg_hbm, deg_v)
        pltpu.sync_copy(src_hbm.at[wid], src_v)
        pltpu.sync_copy(dst_hbm.at[wid], dst_v)
        plsc.subcore_barrier()

        ones16 = jnp.full((16,), 1.0, jnp.float32)

        def body(j, carry):
            pltpu.async_copy(h_hbm.at[src_v.at[j]], rows_v, sem).wait()
            pltpu.sync_copy(rows_v, acc_sh.at[dst_v.at[j]], add=True)
            for k in range(_L // 16):
                idx16 = dst_v[j, pl.ds(k * 16, 16)]
                plsc.addupdate_scatter(deg_v, [idx16], ones16)
            return carry

        lax.fori_loop(0, kb, body, 0)
        plsc.subcore_barrier()
        pltpu.sync_copy(acc_sh.at[pl.ds(s * rpt, rpt)],
                        acc_out.at[c, pl.ds(s * rpt, rpt)])
        pltpu.sync_copy(deg_v, deg_out.at[pl.ds(wid * n_pad, n_pad)])

    return edge_agg


def _dense_layer_body(x_ref, p0_ref, p1_ref, degp_ref, wroot_ref, wrel_ref,
                      b_ref, h_ref):
    deg = jnp.sum(degp_ref[...], axis=0)
    d = jnp.maximum(deg, 1.0)
    agg = (p0_ref[...] + p1_ref[...]) / d[:, None]
    h = (jnp.dot(x_ref[...], wroot_ref[...], preferred_element_type=jnp.float32)
         + jnp.dot(agg, wrel_ref[...], preferred_element_type=jnp.float32)
         + b_ref[...])
    h_ref[...] = jnp.maximum(h, 0.0)


def _dense_pool_body(n_real, h1_ref, p0_ref, p1_ref, degp_ref, wroot_ref,
                     wrel_ref, b_ref, attw_ref, out_ref):
    deg = jnp.sum(degp_ref[...], axis=0)
    d = jnp.maximum(deg, 1.0)
    agg = (p0_ref[...] + p1_ref[...]) / d[:, None]
    h2 = (jnp.dot(h1_ref[...], wroot_ref[...], preferred_element_type=jnp.float32)
          + jnp.dot(agg, wrel_ref[...], preferred_element_type=jnp.float32)
          + b_ref[...])
    rows = lax.broadcasted_iota(jnp.int32, h2.shape, 0)
    h2 = jnp.where(rows < n_real, h2, 0.0)
    m = jnp.sum(h2, axis=0, keepdims=True) / float(n_real)
    gc = jnp.tanh(jnp.dot(m, attw_ref[...],
                          preferred_element_type=jnp.float32))
    scores = jax.nn.sigmoid(jnp.sum(h2 * gc, axis=1, keepdims=True))
    out_ref[...] = jnp.sum(h2 * scores, axis=0, keepdims=True)


def kernel(x, edge_index, W1_root, W1_rel, b1, W2_root, W2_rel, b2, att_W):
    n, d = x.shape
    e = edge_index.shape[1]
    n_pad = ((n + 1 + 127) // 128) * 128
    kb = -(-e // (_NW * _L))
    e_pad = _NW * _L * kb

    src = edge_index[0]
    dst = edge_index[1]
    pad = e_pad - e
    src_p = jnp.concatenate([src, jnp.zeros((pad,), jnp.int32)]).reshape(_NW, kb, _L)
    dst_p = jnp.concatenate([dst, jnp.full((pad,), n, jnp.int32)]).reshape(_NW, kb, _L)
    xp = jnp.concatenate([x, jnp.zeros((n_pad - n, d), x.dtype)], axis=0)
    zrow = jnp.zeros((n_pad, d), jnp.float32)
    zdeg = jnp.zeros((n_pad,), jnp.float32)

    edge_agg = _make_edge_agg(n_pad, kb, d)

    parts1, degp = edge_agg(xp, src_p, dst_p, zrow, zdeg)
    degp = degp.reshape(_NW, n_pad)
    h1 = pl.pallas_call(
        _dense_layer_body,
        out_shape=jax.ShapeDtypeStruct((n_pad, d), jnp.float32),
    )(xp, parts1[0], parts1[1], degp, W1_root, W1_rel, b1.reshape(1, d))

    parts2, _ = edge_agg(h1, src_p, dst_p, zrow, zdeg)
    pooled = pl.pallas_call(
        functools.partial(_dense_pool_body, n),
        out_shape=jax.ShapeDtypeStruct((1, d), jnp.float32),
    )(h1, parts2[0], parts2[1], degp, W2_root, W2_rel, b2.reshape(1, d), att_W)
    return pooled.reshape(d)
